# baseline (speedup 1.0000x reference)
import functools

import jax
import jax.numpy as jnp
from jax import lax
from jax.experimental import pallas as pl
from jax.experimental.pallas import tpu as pltpu
from jax.experimental.pallas import tpu_sc as plsc

B, T, D = 4, 8192, 768
PAD = 100000
NC, NS = 2, 16
NW = NC * NS
PW = T // NW
C = 64
NCH = PW // C
KV = D // 16


_mesh = plsc.VectorSubcoreMesh(core_axis_name="c", subcore_axis_name="s")


@functools.partial(
    pl.kernel,
    out_type=jax.ShapeDtypeStruct((B * T, D), jnp.float32),
    mesh=_mesh,
    scratch_types=[
        pltpu.VMEM((C,), jnp.int32),
        pltpu.VMEM((C,), jnp.int32),
        pltpu.VMEM((C,), jnp.float32),
        pltpu.VMEM((C, D), jnp.float32),
        pltpu.VMEM((C, D), jnp.float32),
        pltpu.SemaphoreType.DMA,
    ],
)
def _emb_lookup(x_hbm, emb_hbm, pos_hbm, out_hbm,
                idxraw, idxsafe, maskbuf, posbuf, ebuf, sem):
    wid = lax.axis_index("s") * NC + lax.axis_index("c")
    pos_base = wid * PW

    def step(it, carry):
        pc = it // B
        b = it % B
        t0 = pos_base + pc * C
        flat0 = b * T + t0

        @pl.when(b == 0)
        def _():
            pltpu.sync_copy(pos_hbm.at[pl.ds(t0, C)], posbuf)

        pltpu.sync_copy(x_hbm.at[pl.ds(flat0, C)], idxraw)
        for k in range(C // 16):
            sl = pl.ds(k * 16, 16)
            v = idxraw[sl]
            ispad = v == PAD
            idxsafe[sl] = jnp.where(ispad, 0, v)
            maskbuf[sl] = jnp.where(ispad, 0.0, 1.0)

        pltpu.async_copy(emb_hbm.at[idxsafe], ebuf, sem).wait()

        for g in range(C // 16):
            mv = maskbuf[pl.ds(g * 16, 16)]

            def row(rr, c2, g=g, mv=mv):
                r = g * 16 + rr
                m = lax.gather(
                    mv, jnp.full((16, 1), rr, jnp.int32),
                    lax.GatherDimensionNumbers(
                        offset_dims=(), collapsed_slice_dims=(0,),
                        start_index_map=(0,)),
                    (1,), mode=lax.GatherScatterMode.PROMISE_IN_BOUNDS)
                for k in range(KV):
                    sl = pl.ds(k * 16, 16)
                    ebuf[r, sl] = ebuf[r, sl] * m + posbuf[r, sl]
                return c2

            lax.fori_loop(0, 16, row, 0)
        pltpu.sync_copy(ebuf, out_hbm.at[pl.ds(flat0, C)])
        return carry

    lax.fori_loop(0, NCH * B, step, 0)


def kernel(x, emb_table, pos_table):
    out = _emb_lookup(x.reshape(-1).astype(jnp.int32), emb_table, pos_table)
    return out.reshape(B, T, D)
